# baseline (speedup 1.0000x reference)
import jax
import jax.numpy as jnp
from jax import lax
from jax.experimental import pallas as pl
from jax.experimental.pallas import tpu as pltpu
from jax.experimental.pallas import tpu_sc as plsc

BATCH = 4096
HIST = 50
EMBED_DIM = 64

NUM_CORES = 2
NUM_SUBCORES = 16
NUM_WORKERS = NUM_CORES * NUM_SUBCORES
CHUNK = 128
N_IDX = BATCH * HIST
PER_WORKER = N_IDX // NUM_WORKERS
N_CHUNKS = PER_WORKER // CHUNK


def _body(x_hbm, table_hbm, out_hbm, idx_v, rows0, rows1, gsem, osem):
    c = lax.axis_index("c")
    s = lax.axis_index("s")
    wid = s * NUM_CORES + c

    pltpu.sync_copy(x_hbm.at[wid], idx_v)

    def _add1(j, carry):
        for i in range(CHUNK // 16):
            sl = pl.ds(i * 16, 16)
            idx_v[j, sl] = idx_v[j, sl] + 1
        return carry

    lax.fori_loop(0, N_CHUNKS, _add1, 0)

    pltpu.async_copy(table_hbm.at[idx_v.at[0]], rows0, gsem)

    def _pair(p, carry):
        j0 = 2 * p
        pltpu.make_async_copy(table_hbm.at[idx_v.at[j0]], rows0, gsem).wait()
        pltpu.async_copy(table_hbm.at[idx_v.at[j0 + 1]], rows1, gsem)
        pltpu.async_copy(rows0, out_hbm.at[wid, j0], osem)
        pltpu.make_async_copy(table_hbm.at[idx_v.at[j0 + 1]], rows1, gsem).wait()
        pltpu.make_async_copy(rows0, out_hbm.at[wid, j0], osem).wait()

        @pl.when(p + 1 < N_CHUNKS // 2)
        def _():
            pltpu.async_copy(table_hbm.at[idx_v.at[j0 + 2]], rows0, gsem)

        pltpu.async_copy(rows1, out_hbm.at[wid, j0 + 1], osem)
        pltpu.make_async_copy(rows1, out_hbm.at[wid, j0 + 1], osem).wait()
        return carry

    lax.fori_loop(0, N_CHUNKS // 2, _pair, 0)


def _sc_gather(idx, table):
    mesh = plsc.VectorSubcoreMesh(
        core_axis_name="c",
        subcore_axis_name="s",
        num_cores=NUM_CORES,
        num_subcores=NUM_SUBCORES,
    )
    return pl.kernel(
        _body,
        out_type=jax.ShapeDtypeStruct(
            (NUM_WORKERS, N_CHUNKS, CHUNK, EMBED_DIM), jnp.float32
        ),
        mesh=mesh,
        scratch_types=[
            pltpu.VMEM((N_CHUNKS, CHUNK), jnp.int32),
            pltpu.VMEM((CHUNK, EMBED_DIM), jnp.float32),
            pltpu.VMEM((CHUNK, EMBED_DIM), jnp.float32),
            pltpu.SemaphoreType.DMA,
            pltpu.SemaphoreType.DMA,
        ],
        compiler_params=pltpu.CompilerParams(use_tc_tiling_on_sc=False),
    )(idx, table)


def kernel(x, table):
    idx = x.astype(jnp.int32).reshape(NUM_WORKERS, N_CHUNKS, CHUNK)
    out = _sc_gather(idx, table)
    return out.reshape(BATCH, HIST, EMBED_DIM)
